# baseline (speedup 1.0000x reference)
import functools

import jax
import jax.numpy as jnp
from jax import lax
from jax.experimental import pallas as pl
from jax.experimental.pallas import tpu as pltpu
from jax.experimental.pallas import tpu_sc as plsc

NUM_CATEGORIES = 100000
N = 8388608

_NC = 2
_NS = 16
_NW = _NC * _NS
_PER = N // _NW
_CHUNK = 4096
_NBUF = 3
_NCHUNK = _PER // _CHUNK
_L = 16
_HPAD = 100000
_UNROLL = 8


def _sc_hist_body(data_hbm, weights_hbm, out_hbm, wsum_hbm, hist, sumv,
                  d0, d1, d2, w0, w1, w2, *sems):
    dbufs = (d0, d1, d2)
    wbufs = (w0, w1, w2)
    wid = lax.axis_index("s") * _NC + lax.axis_index("c")
    base = wid * _PER

    def start_fetch(slot, c):
        off = base + c * _CHUNK
        pltpu.async_copy(data_hbm.at[pl.ds(off, _CHUNK)],
                         dbufs[slot], sems[2 * slot])
        pltpu.async_copy(weights_hbm.at[pl.ds(off, _CHUNK)],
                         wbufs[slot], sems[2 * slot + 1])

    def wait_fetch(slot):
        pltpu.make_async_copy(data_hbm.at[pl.ds(0, _CHUNK)], dbufs[slot],
                              sems[2 * slot]).wait()
        pltpu.make_async_copy(weights_hbm.at[pl.ds(0, _CHUNK)], wbufs[slot],
                              sems[2 * slot + 1]).wait()

    def scatter_chunk(slot, accs):
        @plsc.parallel_loop(0, _CHUNK, step=_L, unroll=_UNROLL, carry=accs)
        def _scatter_body(i, a):
            s = pl.ds(i, _L)
            w = wbufs[slot][s]
            plsc.addupdate_scatter(hist, [dbufs[slot][s]], w)
            return a[1:] + (a[0] + w,)

        return _scatter_body

    for b in range(_NBUF):
        start_fetch(b, b)

    zeros = jnp.zeros((_L,), jnp.float32)

    @plsc.parallel_loop(0, _HPAD, step=_L, unroll=10)
    def _zero_body(i):
        hist[pl.ds(i, _L)] = zeros

    _NFULL = (_NCHUNK - _NBUF) // _NBUF - 1

    def ring_body(t, accs):
        c = _NBUF * t
        for b in range(_NBUF):
            wait_fetch(b)
            accs = scatter_chunk(b, accs)
            start_fetch(b, c + _NBUF + b)
        return accs

    accs = tuple(jnp.zeros((_L,), jnp.float32) for _ in range(8))
    accs = lax.fori_loop(0, _NFULL + 1, ring_body, accs)

    wait_fetch(0)
    accs = scatter_chunk(0, accs)
    start_fetch(0, _NCHUNK - 1)
    for b in range(1, _NBUF):
        wait_fetch(b)
        accs = scatter_chunk(b, accs)
    wait_fetch(0)
    accs = scatter_chunk(0, accs)

    acc = ((accs[0] + accs[1]) + (accs[2] + accs[3])) + \
          ((accs[4] + accs[5]) + (accs[6] + accs[7]))
    sumv[...] = acc
    pltpu.sync_copy(hist, out_hbm.at[wid])
    pltpu.sync_copy(sumv, wsum_hbm.at[wid])


_sc_hist = functools.partial(
    pl.kernel,
    out_type=(jax.ShapeDtypeStruct((_NW, _HPAD), jnp.float32),
              jax.ShapeDtypeStruct((_NW, _L), jnp.float32)),
    mesh=plsc.VectorSubcoreMesh(core_axis_name="c", subcore_axis_name="s"),
    compiler_params=pltpu.CompilerParams(needs_layout_passes=False),
    scratch_types=[
        pltpu.VMEM((_HPAD,), jnp.float32),
        pltpu.VMEM((_L,), jnp.float32),
    ] + [pltpu.VMEM((_CHUNK,), jnp.int32)] * _NBUF
      + [pltpu.VMEM((_CHUNK,), jnp.float32)] * _NBUF
      + [pltpu.SemaphoreType.DMA] * (2 * _NBUF),
)(_sc_hist_body)


_TCB = 16384


def _tc_reduce_body(parts_ref, wsums_ref, out_ref, tot_ref):
    @pl.when(pl.program_id(0) == 0)
    def _():
        tot_ref[0] = jnp.sum(wsums_ref[...])

    s = jnp.sum(parts_ref[...], axis=0)
    out_ref[...] = jnp.log(s / tot_ref[0])


_tc_reduce = pl.pallas_call(
    _tc_reduce_body,
    grid=((NUM_CATEGORIES + _TCB - 1) // _TCB,),
    in_specs=[
        pl.BlockSpec((_NW, _TCB), lambda g: (0, g)),
        pl.BlockSpec((_NW, _L), lambda g: (0, 0)),
    ],
    out_specs=pl.BlockSpec((_TCB,), lambda g: (g,)),
    out_shape=jax.ShapeDtypeStruct((NUM_CATEGORIES,), jnp.float32),
    scratch_shapes=[pltpu.SMEM((1,), jnp.float32)],
)


def kernel(data, weights):
    parts, wsums = _sc_hist(data, weights)
    return _tc_reduce(parts, wsums)


# speedup vs baseline: 1.0117x; 1.0117x over previous
import functools

import jax
import jax.numpy as jnp
from jax import lax
from jax.experimental import pallas as pl
from jax.experimental.pallas import tpu as pltpu
from jax.experimental.pallas import tpu_sc as plsc

NUM_CATEGORIES = 100000
N = 8388608

_NC = 2
_NS = 16
_NW = _NC * _NS
_PER = N // _NW
_CHUNK = 4096
_NBUF = 3
_NCHUNK = _PER // _CHUNK
_L = 16
_HPAD = 100000
_UNROLL = 8


def _sc_hist_body(data_hbm, weights_hbm, out_hbm, wsum_hbm, hist, sumv,
                  d0, d1, d2, w0, w1, w2, *sems):
    dbufs = (d0, d1, d2)
    wbufs = (w0, w1, w2)
    wid = lax.axis_index("s") * _NC + lax.axis_index("c")
    base = wid * _PER

    def start_fetch(slot, c):
        off = base + c * _CHUNK
        pltpu.async_copy(data_hbm.at[pl.ds(off, _CHUNK)],
                         dbufs[slot], sems[2 * slot])
        pltpu.async_copy(weights_hbm.at[pl.ds(off, _CHUNK)],
                         wbufs[slot], sems[2 * slot + 1])

    def wait_fetch(slot):
        pltpu.make_async_copy(data_hbm.at[pl.ds(0, _CHUNK)], dbufs[slot],
                              sems[2 * slot]).wait()
        pltpu.make_async_copy(weights_hbm.at[pl.ds(0, _CHUNK)], wbufs[slot],
                              sems[2 * slot + 1]).wait()

    def scatter_chunk(slot, accs):
        @plsc.parallel_loop(0, _CHUNK, step=_L, unroll=_UNROLL, carry=accs)
        def _scatter_body(i, a):
            s = pl.ds(i, _L)
            w = wbufs[slot][s]
            plsc.addupdate_scatter(hist, [dbufs[slot][s]], w)
            return a[1:] + (a[0] + w,)

        return _scatter_body

    for b in range(_NBUF):
        start_fetch(b, b)

    zeros = jnp.zeros((_L,), jnp.float32)

    @plsc.parallel_loop(0, _HPAD, step=_L, unroll=10)
    def _zero_body(i):
        hist[pl.ds(i, _L)] = zeros

    _NFULL = (_NCHUNK - _NBUF) // _NBUF - 1

    def ring_body(t, accs):
        c = _NBUF * t
        for b in range(_NBUF):
            wait_fetch(b)
            accs = scatter_chunk(b, accs)
            start_fetch(b, c + _NBUF + b)
        return accs

    accs = tuple(jnp.zeros((_L,), jnp.float32) for _ in range(8))
    accs = lax.fori_loop(0, _NFULL + 1, ring_body, accs)

    wait_fetch(0)
    accs = scatter_chunk(0, accs)
    start_fetch(0, _NCHUNK - 1)
    for b in range(1, _NBUF):
        wait_fetch(b)
        accs = scatter_chunk(b, accs)
    wait_fetch(0)
    accs = scatter_chunk(0, accs)

    acc = ((accs[0] + accs[1]) + (accs[2] + accs[3])) + \
          ((accs[4] + accs[5]) + (accs[6] + accs[7]))
    sumv[...] = acc
    pltpu.sync_copy(hist, out_hbm.at[wid])
    pltpu.sync_copy(sumv, wsum_hbm.at[wid])


_sc_hist = functools.partial(
    pl.kernel,
    out_type=(jax.ShapeDtypeStruct((_NW, _HPAD), jnp.float32),
              jax.ShapeDtypeStruct((_NW, _L), jnp.float32)),
    mesh=plsc.VectorSubcoreMesh(core_axis_name="c", subcore_axis_name="s"),
    compiler_params=pltpu.CompilerParams(needs_layout_passes=False),
    scratch_types=[
        pltpu.VMEM((_HPAD,), jnp.float32),
        pltpu.VMEM((_L,), jnp.float32),
    ] + [pltpu.VMEM((_CHUNK,), jnp.int32)] * _NBUF
      + [pltpu.VMEM((_CHUNK,), jnp.float32)] * _NBUF
      + [pltpu.SemaphoreType.DMA] * (2 * _NBUF),
)(_sc_hist_body)


_TCB = 32768


def _tc_reduce_body(parts_ref, wsums_ref, out_ref, tot_ref):
    @pl.when(pl.program_id(0) == 0)
    def _():
        tot_ref[0] = jnp.sum(wsums_ref[...])

    s = jnp.sum(parts_ref[...], axis=0)
    out_ref[...] = jnp.log(s / tot_ref[0])


_tc_reduce = pl.pallas_call(
    _tc_reduce_body,
    grid=((NUM_CATEGORIES + _TCB - 1) // _TCB,),
    in_specs=[
        pl.BlockSpec((_NW, _TCB), lambda g: (0, g)),
        pl.BlockSpec((_NW, _L), lambda g: (0, 0)),
    ],
    out_specs=pl.BlockSpec((_TCB,), lambda g: (g,)),
    out_shape=jax.ShapeDtypeStruct((NUM_CATEGORIES,), jnp.float32),
    scratch_shapes=[pltpu.SMEM((1,), jnp.float32)],
)


def kernel(data, weights):
    parts, wsums = _sc_hist(data, weights)
    return _tc_reduce(parts, wsums)


# speedup vs baseline: 1.0181x; 1.0063x over previous
import functools

import jax
import jax.numpy as jnp
from jax import lax
from jax.experimental import pallas as pl
from jax.experimental.pallas import tpu as pltpu
from jax.experimental.pallas import tpu_sc as plsc

NUM_CATEGORIES = 100000
N = 8388608

_NC = 2
_NS = 16
_NW = _NC * _NS
_PER = N // _NW
_CHUNK = 4096
_NBUF = 3
_NCHUNK = _PER // _CHUNK
_L = 16
_HPAD = 100000
_UNROLL = 8


def _sc_hist_body(data_hbm, weights_hbm, out_hbm, wsum_hbm, hist, sumv,
                  d0, d1, d2, w0, w1, w2, *sems):
    dbufs = (d0, d1, d2)
    wbufs = (w0, w1, w2)
    wid = lax.axis_index("s") * _NC + lax.axis_index("c")
    base = wid * _PER

    def start_fetch(slot, c):
        off = base + c * _CHUNK
        pltpu.async_copy(data_hbm.at[pl.ds(off, _CHUNK)],
                         dbufs[slot], sems[2 * slot])
        pltpu.async_copy(weights_hbm.at[pl.ds(off, _CHUNK)],
                         wbufs[slot], sems[2 * slot + 1])

    def wait_fetch(slot):
        pltpu.make_async_copy(data_hbm.at[pl.ds(0, _CHUNK)], dbufs[slot],
                              sems[2 * slot]).wait()
        pltpu.make_async_copy(weights_hbm.at[pl.ds(0, _CHUNK)], wbufs[slot],
                              sems[2 * slot + 1]).wait()

    def scatter_chunk(slot, accs):
        @plsc.parallel_loop(0, _CHUNK, step=_L, unroll=_UNROLL, carry=accs)
        def _scatter_body(i, a):
            s = pl.ds(i, _L)
            w = wbufs[slot][s]
            plsc.addupdate_scatter(hist, [dbufs[slot][s]], w)
            return a[1:] + (a[0] + w,)

        return _scatter_body

    for b in range(_NBUF):
        start_fetch(b, b)

    zeros = jnp.zeros((_L,), jnp.float32)

    @plsc.parallel_loop(0, _HPAD, step=_L, unroll=10)
    def _zero_body(i):
        hist[pl.ds(i, _L)] = zeros

    _NFULL = (_NCHUNK - _NBUF) // _NBUF - 1

    def ring_body(t, accs):
        c = _NBUF * t
        for b in range(_NBUF):
            wait_fetch(b)
            accs = scatter_chunk(b, accs)
            start_fetch(b, c + _NBUF + b)
        return accs

    accs = tuple(jnp.zeros((_L,), jnp.float32) for _ in range(8))
    accs = lax.fori_loop(0, _NFULL + 1, ring_body, accs)

    wait_fetch(0)
    accs = scatter_chunk(0, accs)
    start_fetch(0, _NCHUNK - 1)
    for b in range(1, _NBUF):
        wait_fetch(b)
        accs = scatter_chunk(b, accs)
    wait_fetch(0)
    accs = scatter_chunk(0, accs)

    acc = ((accs[0] + accs[1]) + (accs[2] + accs[3])) + \
          ((accs[4] + accs[5]) + (accs[6] + accs[7]))
    sumv[...] = acc
    pltpu.sync_copy(hist, out_hbm.at[wid])
    pltpu.sync_copy(sumv, wsum_hbm.at[wid])


_sc_hist = functools.partial(
    pl.kernel,
    out_type=(jax.ShapeDtypeStruct((_NW, _HPAD), jnp.float32),
              jax.ShapeDtypeStruct((_NW, _L), jnp.float32)),
    mesh=plsc.VectorSubcoreMesh(core_axis_name="c", subcore_axis_name="s"),
    compiler_params=pltpu.CompilerParams(needs_layout_passes=False),
    scratch_types=[
        pltpu.VMEM((_HPAD,), jnp.float32),
        pltpu.VMEM((_L,), jnp.float32),
    ] + [pltpu.VMEM((_CHUNK,), jnp.int32)] * _NBUF
      + [pltpu.VMEM((_CHUNK,), jnp.float32)] * _NBUF
      + [pltpu.SemaphoreType.DMA] * (2 * _NBUF),
)(_sc_hist_body)


_TCB = 49152


def _tc_reduce_body(parts_ref, wsums_ref, out_ref, tot_ref):
    @pl.when(pl.program_id(0) == 0)
    def _():
        tot_ref[0] = jnp.sum(wsums_ref[...])

    s = jnp.sum(parts_ref[...], axis=0)
    out_ref[...] = jnp.log(s / tot_ref[0])


_tc_reduce = pl.pallas_call(
    _tc_reduce_body,
    grid=((NUM_CATEGORIES + _TCB - 1) // _TCB,),
    in_specs=[
        pl.BlockSpec((_NW, _TCB), lambda g: (0, g)),
        pl.BlockSpec((_NW, _L), lambda g: (0, 0)),
    ],
    out_specs=pl.BlockSpec((_TCB,), lambda g: (g,)),
    out_shape=jax.ShapeDtypeStruct((NUM_CATEGORIES,), jnp.float32),
    scratch_shapes=[pltpu.SMEM((1,), jnp.float32)],
)


def kernel(data, weights):
    parts, wsums = _sc_hist(data, weights)
    return _tc_reduce(parts, wsums)
